# baseline (speedup 1.0000x reference)
import functools
import math

import jax
import jax.numpy as jnp
from jax import lax
from jax.experimental import pallas as pl
from jax.experimental.pallas import tpu as pltpu, tpu_sc as plsc

D = 128
SCALE = float(math.sqrt(D))
NBUF = 5
LOOK = 4

_info = plsc.get_sparse_core_info()
NC, NS, L = _info.num_cores, _info.num_subcores, _info.num_lanes
NW = NC * NS


def _make_lookup(S: int, N: int):
    n_per_w = N // NW

    mesh = plsc.VectorSubcoreMesh(core_axis_name="c", subcore_axis_name="s")

    @functools.partial(
        pl.kernel,
        out_type=jax.ShapeDtypeStruct((S, N, D), jnp.float32),
        mesh=mesh,
        scratch_types=[
            pltpu.VMEM((S, n_per_w), jnp.int32),
            *[pltpu.VMEM((n_per_w, D), jnp.float32) for _ in range(NBUF)],
            *[pltpu.SemaphoreType.DMA for _ in range(NBUF)],
            *[pltpu.SemaphoreType.DMA for _ in range(NBUF)],
        ],
    )
    def lookup_kernel(xt_hbm, table_hbm, out_hbm, idx_v, *rest):
        rows = rest[:NBUF]
        gsem = rest[NBUF:2 * NBUF]
        osem = rest[2 * NBUF:]
        wid = lax.axis_index("s") * NC + lax.axis_index("c")
        n0 = wid * n_per_w

        pltpu.sync_copy(xt_hbm.at[:, pl.ds(n0, n_per_w)], idx_v)

        def start_gather(s, b):
            pltpu.async_copy(table_hbm.at[idx_v.at[s]], rows[b], gsem[b])

        def wait_gather(b):
            pltpu.make_async_copy(
                table_hbm.at[pl.ds(0, n_per_w)], rows[b], gsem[b]).wait()

        def start_out(s, b):
            pltpu.async_copy(
                rows[b], out_hbm.at[s, pl.ds(n0, n_per_w)], osem[b])

        def wait_out(b):
            pltpu.make_async_copy(
                table_hbm.at[pl.ds(0, n_per_w)], rows[b], osem[b]).wait()

        def scale_buf(b):
            buf = rows[b]

            def row_body(r, _):
                for c in range(D // L):
                    sl = pl.ds(c * L, L)
                    buf[r, sl] = buf[r, sl] * SCALE
                return 0

            lax.fori_loop(0, n_per_w, row_body, 0, unroll=4)

        for b in range(LOOK):
            start_gather(b, b)

        def step(i, _):
            for k in range(NBUF):
                s = i * NBUF + k
                wait_gather(k)
                scale_buf(k)
                start_out(s, k)
                kb = (k + LOOK) % NBUF

                @pl.when(s + LOOK < S)
                def _():
                    @pl.when(s >= NBUF - LOOK)
                    def _():
                        wait_out(kb)
                    start_gather(s + LOOK, kb)
            return 0

        lax.fori_loop(0, S // NBUF, step, 0)

        for b in range(NBUF):
            wait_out(b)

    return lookup_kernel


def kernel(x, table):
    n, s = x.shape
    xt = jnp.transpose(x).astype(jnp.int32)
    out_t = _make_lookup(s, n)(xt, table)
    return jnp.transpose(out_t, (1, 0, 2))


# speedup vs baseline: 1.0024x; 1.0024x over previous
import functools
import math

import jax
import jax.numpy as jnp
from jax import lax
from jax.experimental import pallas as pl
from jax.experimental.pallas import tpu as pltpu, tpu_sc as plsc

D = 128
SCALE = float(math.sqrt(D))
NBUF = 5
LOOK = 3

_info = plsc.get_sparse_core_info()
NC, NS, L = _info.num_cores, _info.num_subcores, _info.num_lanes
NW = NC * NS


def _make_lookup(S: int, N: int):
    n_per_w = N // NW

    mesh = plsc.VectorSubcoreMesh(core_axis_name="c", subcore_axis_name="s")

    @functools.partial(
        pl.kernel,
        out_type=jax.ShapeDtypeStruct((S, N, D), jnp.float32),
        mesh=mesh,
        scratch_types=[
            pltpu.VMEM((S, n_per_w), jnp.int32),
            *[pltpu.VMEM((n_per_w, D), jnp.float32) for _ in range(NBUF)],
            *[pltpu.SemaphoreType.DMA for _ in range(NBUF)],
            *[pltpu.SemaphoreType.DMA for _ in range(NBUF)],
        ],
    )
    def lookup_kernel(xt_hbm, table_hbm, out_hbm, idx_v, *rest):
        rows = rest[:NBUF]
        gsem = rest[NBUF:2 * NBUF]
        osem = rest[2 * NBUF:]
        wid = lax.axis_index("s") * NC + lax.axis_index("c")
        n0 = wid * n_per_w

        pltpu.sync_copy(xt_hbm.at[:, pl.ds(n0, n_per_w)], idx_v)

        def start_gather(s, b):
            pltpu.async_copy(table_hbm.at[idx_v.at[s]], rows[b], gsem[b])

        def wait_gather(b):
            pltpu.make_async_copy(
                table_hbm.at[pl.ds(0, n_per_w)], rows[b], gsem[b]).wait()

        def start_out(s, b):
            pltpu.async_copy(
                rows[b], out_hbm.at[s, pl.ds(n0, n_per_w)], osem[b])

        def wait_out(b):
            pltpu.make_async_copy(
                table_hbm.at[pl.ds(0, n_per_w)], rows[b], osem[b]).wait()

        def scale_buf(b):
            buf = rows[b]

            def row_body(r, _):
                for c in range(D // L):
                    sl = pl.ds(c * L, L)
                    buf[r, sl] = buf[r, sl] * SCALE
                return 0

            lax.fori_loop(0, n_per_w, row_body, 0, unroll=4)

        for b in range(LOOK):
            start_gather(b, b)

        def step(i, _):
            for k in range(NBUF):
                s = i * NBUF + k
                wait_gather(k)
                scale_buf(k)
                start_out(s, k)
                kb = (k + LOOK) % NBUF

                @pl.when(s + LOOK < S)
                def _():
                    @pl.when(s >= NBUF - LOOK)
                    def _():
                        wait_out(kb)
                    start_gather(s + LOOK, kb)
            return 0

        lax.fori_loop(0, S // NBUF, step, 0)

        for b in range(NBUF):
            wait_out(b)

    return lookup_kernel


def kernel(x, table):
    n, s = x.shape
    xt = jnp.transpose(x).astype(jnp.int32)
    out_t = _make_lookup(s, n)(xt, table)
    return jnp.transpose(out_t, (1, 0, 2))
